# baseline (speedup 1.0000x reference)
import functools

import jax
import jax.numpy as jnp
from jax import lax
from jax.experimental import pallas as pl
from jax.experimental.pallas import tpu as pltpu
from jax.experimental.pallas import tpu_sc as plsc

_NC = 2
_NS = 16
_NW = _NC * _NS
_L = 16


@functools.lru_cache(maxsize=None)
def _make_gather(vocab, dim, batch, seq):
    bw = batch // _NW
    assert batch % _NW == 0 and bw % 128 == 0 and bw <= 128
    assert dim % _L == 0 and seq % 2 == 0
    row = 2 * dim
    mesh = plsc.VectorSubcoreMesh(core_axis_name="c", subcore_axis_name="s")

    @functools.partial(
        pl.kernel,
        out_type=jax.ShapeDtypeStruct((seq, dim, batch), jnp.float32),
        mesh=mesh,
        scratch_types=[
            pltpu.VMEM((seq, bw), jnp.int32),
            pltpu.VMEM((2, bw, row), jnp.float32),
            pltpu.VMEM((2, dim, bw), jnp.float32),
            pltpu.SemaphoreType.DMA,
            pltpu.SemaphoreType.DMA,
        ],
        compiler_params=pltpu.CompilerParams(needs_layout_passes=False),
    )
    def gather_kernel(xt_hbm, table_hbm, out_hbm, idx_v, rows_v, tr_v, gsem, wsem):
        wid = lax.axis_index("s") * _NC + lax.axis_index("c")
        b0 = wid * bw
        pltpu.sync_copy(xt_hbm.at[:, pl.ds(b0, bw)], idx_v)
        iota = lax.iota(jnp.int32, _L)

        pltpu.async_copy(table_hbm.at[idx_v.at[0]], rows_v.at[0], gsem)

        @pl.loop(0, seq, step=2)
        def _task(t0):
            for h in range(2):
                t = t0 + h

                @pl.when(t + 1 < seq)
                def _():
                    pltpu.async_copy(
                        table_hbm.at[idx_v.at[t + 1]], rows_v.at[1 - h], gsem
                    )

                pltpu.make_async_copy(
                    table_hbm.at[idx_v.at[0]], rows_v.at[h], gsem
                ).wait()

                @pl.when(t0 > 0)
                def _():
                    pltpu.make_async_copy(
                        tr_v.at[h], out_hbm.at[0, :, pl.ds(0, bw)], wsem
                    ).wait()

                rows = rows_v.at[h]
                tr = tr_v.at[h]

                @pl.loop(0, dim)
                def _col(d):
                    col = jnp.full((_L,), d, jnp.int32)
                    for j in range(bw // _L):
                        v = plsc.load_gather(rows, [j * _L + iota, col])
                        tr.at[d][pl.ds(j * _L, _L)] = v

                pltpu.async_copy(tr_v.at[h], out_hbm.at[t, :, pl.ds(b0, bw)], wsem)

        for h in range(2):
            pltpu.make_async_copy(
                tr_v.at[h], out_hbm.at[0, :, pl.ds(0, bw)], wsem
            ).wait()

    return gather_kernel


def kernel(x, embeddings):
    batch, seq = x.shape
    vocab, dim = embeddings.shape
    xt = jnp.transpose(x.astype(jnp.int32))
    padded = jnp.pad(embeddings, ((0, 0), (0, dim)))
    out = _make_gather(vocab, dim, batch, seq)(xt, padded)
    return jnp.transpose(out, (2, 0, 1))
